# baseline (speedup 1.0000x reference)
import functools

import jax
import jax.numpy as jnp
from jax import lax
from jax.experimental import pallas as pl
from jax.experimental.pallas import tpu as pltpu
from jax.experimental.pallas import tpu_sc as plsc

N = 10000
E = 320000
D = 128
ED = 16
S = 100

BN = 400
BE = 512

NC = 2
NS = 16
NW = NC * NS
G = 80

CHUNKS = (51200, 89600, 89600, 89600)


def _silu(x):
    return x * jax.nn.sigmoid(x)


def _embed_body(sp_ref, emb_ref, out_ref):
    sp = sp_ref[...]
    iota = lax.broadcasted_iota(jnp.int32, (BN, S), 1)
    onehot = (sp == iota).astype(jnp.float32)
    out_ref[...] = jnp.dot(onehot, emb_ref[...],
                           preferred_element_type=jnp.float32)


def _embed(species, embed_atom):
    return pl.pallas_call(
        _embed_body,
        grid=(N // BN,),
        in_specs=[
            pl.BlockSpec((BN, 1), lambda n: (n, 0)),
            pl.BlockSpec((S, D), lambda n: (0, 0)),
        ],
        out_specs=pl.BlockSpec((BN, D), lambda n: (n, 0)),
        out_shape=jax.ShapeDtypeStruct((N, D), jnp.float32),
    )(species.reshape(N, 1), embed_atom)


def _sc_gather_body(EPW, NIT,
                    f_hbm, px_hbm, py_hbm, pz_hbm, i_hbm, j_hbm,
                    gi_hbm, gj_hbm, vx_hbm, vy_hbm, vz_hbm,
                    idx_i, idx_j,
                    rows_i0, rows_j0, pbi0, pbj0, vb0,
                    rows_i1, rows_j1, pbi1, pbj1, vb1,
                    gsem0, gsem1, wsem0, wsem1):
    wid = lax.axis_index("s") * NC + lax.axis_index("c")
    base = wid * EPW
    pltpu.sync_copy(i_hbm.at[pl.ds(base, EPW)], idx_i)
    pltpu.sync_copy(j_hbm.at[pl.ds(base, EPW)], idx_j)

    sets = [(rows_i0, rows_j0, pbi0, pbj0, vb0, gsem0, wsem0),
            (rows_i1, rows_j1, pbi1, pbj1, vb1, gsem1, wsem1)]

    def fire_gathers(it, s):
        rows_i, rows_j, pbi, pbj, vb, gsem, wsem = s
        off = it * G
        ii = idx_i.at[pl.ds(off, G)]
        jj = idx_j.at[pl.ds(off, G)]
        pltpu.async_copy(f_hbm.at[ii], rows_i, gsem)
        pltpu.async_copy(f_hbm.at[jj], rows_j, gsem)
        pltpu.async_copy(px_hbm.at[ii], pbi.at[0], gsem)
        pltpu.async_copy(py_hbm.at[ii], pbi.at[1], gsem)
        pltpu.async_copy(pz_hbm.at[ii], pbi.at[2], gsem)
        pltpu.async_copy(px_hbm.at[jj], pbj.at[0], gsem)
        pltpu.async_copy(py_hbm.at[jj], pbj.at[1], gsem)
        pltpu.async_copy(pz_hbm.at[jj], pbj.at[2], gsem)

    def drain_gathers(s):
        rows_i, rows_j, pbi, pbj, vb, gsem, wsem = s
        dsrc = f_hbm.at[pl.ds(0, G)]
        pltpu.make_async_copy(dsrc, rows_i, gsem).wait()
        pltpu.make_async_copy(dsrc, rows_j, gsem).wait()
        dp = px_hbm.at[pl.ds(0, G)]
        for b in (pbi, pbj):
            for q in range(3):
                pltpu.make_async_copy(dp, b.at[q], gsem).wait()

    def compute_and_write(it, s):
        rows_i, rows_j, pbi, pbj, vb, gsem, wsem = s
        off = it * G
        for q in range(3):
            for r in range(G // 16):
                sl = pl.ds(r * 16, 16)
                vb[q, sl] = pbj[q, sl] - pbi[q, sl]
        pltpu.async_copy(rows_i, gi_hbm.at[pl.ds(base + off, G), :], wsem)
        pltpu.async_copy(rows_j, gj_hbm.at[pl.ds(base + off, G), :], wsem)
        pltpu.async_copy(vb.at[0], vx_hbm.at[pl.ds(base + off, G)], wsem)
        pltpu.async_copy(vb.at[1], vy_hbm.at[pl.ds(base + off, G)], wsem)
        pltpu.async_copy(vb.at[2], vz_hbm.at[pl.ds(base + off, G)], wsem)

    def drain_writes(s):
        rows_i, rows_j, pbi, pbj, vb, gsem, wsem = s
        dsrc = f_hbm.at[pl.ds(0, G)]
        pltpu.make_async_copy(dsrc, rows_i, wsem).wait()
        pltpu.make_async_copy(dsrc, rows_j, wsem).wait()
        dp = px_hbm.at[pl.ds(0, G)]
        pltpu.make_async_copy(dp, vb.at[0], wsem).wait()
        pltpu.make_async_copy(dp, vb.at[1], wsem).wait()
        pltpu.make_async_copy(dp, vb.at[2], wsem).wait()

    fire_gathers(0, sets[0])

    def body(h, carry):
        k = 2 * h
        drain_gathers(sets[0])

        @pl.when(h > 0)
        def _():
            drain_writes(sets[1])

        fire_gathers(k + 1, sets[1])
        compute_and_write(k, sets[0])
        drain_gathers(sets[1])
        drain_writes(sets[0])

        @pl.when(k + 2 < NIT)
        def _():
            fire_gathers(k + 2, sets[0])

        compute_and_write(k + 1, sets[1])
        return carry

    lax.fori_loop(0, NIT // 2, body, 0)

    if NIT % 2:
        drain_gathers(sets[0])
        drain_writes(sets[1])
        compute_and_write(NIT - 1, sets[0])
        drain_writes(sets[0])
    else:
        drain_writes(sets[1])


def _sc_gather(f, px, py, pz, i, j):
    ec = i.shape[0]
    EPW = ec // NW
    NIT = EPW // G
    ev = jax.ShapeDtypeStruct((ec,), jnp.float32)
    return pl.kernel(
        functools.partial(_sc_gather_body, EPW, NIT),
        out_type=[jax.ShapeDtypeStruct((ec, D), jnp.float32),
                  jax.ShapeDtypeStruct((ec, D), jnp.float32),
                  ev, ev, ev],
        mesh=plsc.VectorSubcoreMesh(core_axis_name="c", subcore_axis_name="s"),
        scratch_types=[
            pltpu.VMEM((EPW,), jnp.int32),
            pltpu.VMEM((EPW,), jnp.int32),
            pltpu.VMEM((G, D), jnp.float32),
            pltpu.VMEM((G, D), jnp.float32),
            pltpu.VMEM((3, G), jnp.float32),
            pltpu.VMEM((3, G), jnp.float32),
            pltpu.VMEM((3, G), jnp.float32),
            pltpu.VMEM((G, D), jnp.float32),
            pltpu.VMEM((G, D), jnp.float32),
            pltpu.VMEM((3, G), jnp.float32),
            pltpu.VMEM((3, G), jnp.float32),
            pltpu.VMEM((3, G), jnp.float32),
            pltpu.SemaphoreType.DMA,
            pltpu.SemaphoreType.DMA,
            pltpu.SemaphoreType.DMA,
            pltpu.SemaphoreType.DMA,
        ],
    )(f, px, py, pz, i, j)


NP = 10240
NPT = NP // NS


def _sc_scatter_body(NG, NTAIL, GPT,
                     ja_hbm, msga_hbm, msgb_hbm, outa_hbm, outb_hbm,
                     idx8, rows0, rows1, acc, sem0, sem1):
    c = lax.axis_index("c")
    t = lax.axis_index("s")

    def zrow(r, carry):
        for l in range(8):
            rows0[r, pl.ds(l * 16, 16)] = jnp.zeros((16,), jnp.float32)
        return carry

    lax.fori_loop(0, 128, zrow, 0)
    for k in range(NPT // 128):
        pltpu.sync_copy(rows0, acc.at[pl.ds(t * NPT + k * 128, 128), :])
    plsc.subcore_barrier()

    ngroups = (NG - 1 - t) // NS + 1
    nrows = ngroups * 8

    def process(msg_hbm):
        def grow(r):
            return ((r // 8) * NS + t) * 8 + (r % 8)

        def start(r, buf, sem):
            return pltpu.async_copy(
                msg_hbm.at[pl.ds(grow(r) * 128, 128), :], buf, sem)

        def drain(buf, sem):
            pltpu.make_async_copy(msg_hbm.at[pl.ds(0, 128), :], buf, sem).wait()

        def stage_idx(r):
            g = (r // 8) * NS + t
            pltpu.sync_copy(ja_hbm.at[pl.ds(g * 8, 8), :], idx8)

        def scat(r, buf):
            pltpu.sync_copy(buf, acc.at[idx8.at[lax.rem(r, 8)]], add=True)

        start(0, rows0, sem0)
        start(1, rows1, sem1)

        def body(h, carry):
            r = 2 * h

            @pl.when(lax.rem(r, 8) == 0)
            def _():
                stage_idx(r)

            drain(rows0, sem0)
            scat(r, rows0)

            @pl.when(r + 2 < nrows)
            def _():
                start(r + 2, rows0, sem0)

            drain(rows1, sem1)
            scat(r + 1, rows1)

            @pl.when(r + 3 < nrows)
            def _():
                start(r + 3, rows1, sem1)

            return carry

        lax.fori_loop(0, nrows // 2, body, 0)

        if NTAIL:
            @pl.when(t == 0)
            def _():
                pltpu.sync_copy(ja_hbm.at[pl.ds(NG * 8, 8), :], idx8)
                for k in range(NTAIL):
                    pltpu.sync_copy(
                        msg_hbm.at[pl.ds((NG * 8 + k) * 128, 128), :], rows0)
                    pltpu.sync_copy(rows0, acc.at[idx8.at[k]], add=True)

    @pl.when(c == 0)
    def _():
        process(msga_hbm)

    @pl.when(c == 1)
    def _():
        process(msgb_hbm)

    plsc.subcore_barrier()
    sl = pl.ds(t * NPT, NPT)

    @pl.when(c == 0)
    def _():
        pltpu.sync_copy(acc.at[sl, :], outa_hbm.at[sl, :])

    @pl.when(c == 1)
    def _():
        pltpu.sync_copy(acc.at[sl, :], outb_hbm.at[sl, :])


def _sc_scatter2(j3, msga, msgb):
    nr = msga.shape[0] // 128
    NG = nr // 8
    NTAIL = nr - NG * 8
    GPT = (NG + NS - 1) // NS
    on = jax.ShapeDtypeStruct((NP, D), jnp.float32)
    return pl.kernel(
        functools.partial(_sc_scatter_body, NG, NTAIL, GPT),
        out_type=[on, on],
        mesh=plsc.VectorSubcoreMesh(core_axis_name="c", subcore_axis_name="s"),
        scratch_types=[
            pltpu.VMEM((8, 128), jnp.int32),
            pltpu.VMEM((128, D), jnp.float32),
            pltpu.VMEM((128, D), jnp.float32),
            pltpu.VMEM_SHARED((NP, D), jnp.float32),
            pltpu.SemaphoreType.DMA,
            pltpu.SemaphoreType.DMA,
        ],
    )(j3, msga, msgb)


def _edge_body(gi_ref, gj_ref, vec_ref,
               Wb1, bb1, Wb2, bb2,
               Ws1a, Ws1b, Ws1c, bs1, Ws2, bs2,
               Wv1a, Wv1b, Wv1c, bv1, Wv2, bv2,
               ea_ref, msgs_ref, mvx_ref, mvy_ref, mvz_ref):
    fi = gi_ref[...]
    fj = gj_ref[...]
    vec = vec_ref[...][:, :3]
    length = jnp.sqrt(jnp.sum(vec * vec, axis=1, keepdims=True))

    ea = jnp.dot(_silu(length * Wb1[...] + bb1[...]), Wb2[...],
                 preferred_element_type=jnp.float32) + bb2[...]
    ea_ref[...] = ea

    pre_s = (jnp.dot(fi, Ws1a[...], preferred_element_type=jnp.float32)
             + jnp.dot(fj, Ws1b[...], preferred_element_type=jnp.float32)
             + jnp.dot(ea, Ws1c[...], preferred_element_type=jnp.float32)
             + bs1[...])
    msgs_ref[...] = (jnp.dot(_silu(pre_s), Ws2[...],
                             preferred_element_type=jnp.float32) + bs2[...]) * fi

    pre_v = (jnp.dot(fi, Wv1a[...], preferred_element_type=jnp.float32)
             + jnp.dot(fj, Wv1b[...], preferred_element_type=jnp.float32)
             + jnp.dot(ea, Wv1c[...], preferred_element_type=jnp.float32)
             + bv1[...])
    mv = jnp.dot(_silu(pre_v), Wv2[...],
                 preferred_element_type=jnp.float32) + bv2[...]
    mvx_ref[...] = mv * vec[:, 0:1]
    mvy_ref[...] = mv * vec[:, 1:2]
    mvz_ref[...] = mv * vec[:, 2:3]


def _edges(gi, gj, vec, Wb1, bb1, Wb2, bb2, Ws1, bs1, Ws2, bs2, Wv1, bv1, Wv2, bv2):
    ec = gi.shape[0]
    full = lambda shape: pl.BlockSpec(shape, lambda e: (0, 0))
    wspecs = [
        full((1, ED)), full((1, ED)), full((ED, ED)), full((1, ED)),
        full((D, D)), full((D, D)), full((ED, D)), full((1, D)),
        full((D, D)), full((1, D)),
        full((D, D)), full((D, D)), full((ED, D)), full((1, D)),
        full((D, D)), full((1, D)),
    ]
    out_shapes = [
        jax.ShapeDtypeStruct((ec, ED), jnp.float32),
        jax.ShapeDtypeStruct((ec, D), jnp.float32),
        jax.ShapeDtypeStruct((ec, D), jnp.float32),
        jax.ShapeDtypeStruct((ec, D), jnp.float32),
        jax.ShapeDtypeStruct((ec, D), jnp.float32),
    ]
    out_specs = [
        pl.BlockSpec((BE, ED), lambda e: (e, 0)),
        pl.BlockSpec((BE, D), lambda e: (e, 0)),
        pl.BlockSpec((BE, D), lambda e: (e, 0)),
        pl.BlockSpec((BE, D), lambda e: (e, 0)),
        pl.BlockSpec((BE, D), lambda e: (e, 0)),
    ]
    return pl.pallas_call(
        _edge_body,
        grid=(ec // BE,),
        in_specs=[pl.BlockSpec((BE, D), lambda e: (e, 0)),
                  pl.BlockSpec((BE, D), lambda e: (e, 0)),
                  pl.BlockSpec((BE, 4), lambda e: (e, 0))] + wspecs,
        out_specs=out_specs,
        out_shape=out_shapes,
    )(gi, gj, vec,
      Wb1, bb1.reshape(1, ED), Wb2, bb2.reshape(1, ED),
      Ws1[:D], Ws1[D:2 * D], Ws1[2 * D:], bs1.reshape(1, D), Ws2, bs2.reshape(1, D),
      Wv1[:D], Wv1[D:2 * D], Wv1[2 * D:], bv1.reshape(1, D), Wv2, bv2.reshape(1, D))


def _node_body(f_ref, *rest):
    naggs = len(rest) - 6
    aggs = rest[:naggs]
    Wh1a, Wh1b, bh1, Wh2, bh2, out_ref = rest[naggs:]
    agg = aggs[0][...]
    for a in aggs[1:]:
        agg = agg + a[...]
    pre = (jnp.dot(f_ref[...], Wh1a[...], preferred_element_type=jnp.float32)
           + jnp.dot(agg, Wh1b[...], preferred_element_type=jnp.float32)
           + bh1[...])
    out_ref[...] = jnp.dot(_silu(pre), Wh2[...],
                           preferred_element_type=jnp.float32) + bh2[...]


def _nodes(f, aggs, Wh1, bh1, Wh2, bh2):
    full = lambda shape: pl.BlockSpec(shape, lambda n: (0, 0))
    nspec = pl.BlockSpec((BN, D), lambda n: (n, 0))
    return pl.pallas_call(
        _node_body,
        grid=(N // BN,),
        in_specs=[nspec] + [nspec] * len(aggs) + [
            full((D, D)), full((D, D)), full((1, D)),
            full((D, D)), full((1, D))],
        out_specs=pl.BlockSpec((BN, D), lambda n: (n, 0)),
        out_shape=jax.ShapeDtypeStruct((N, D), jnp.float32),
    )(f, *aggs, Wh1[:D], Wh1[D:], bh1.reshape(1, D),
      Wh2, bh2.reshape(1, D))


def kernel(species, pos, edge_index, embed_atom,
           Wb1, bb1, Wb2, bb2,
           Ws1, bs1, Ws2, bs2,
           Wh1, bh1, Wh2, bh2,
           Wv1, bv1, Wv2, bv2):
    i = edge_index[0]
    j = edge_index[1]

    f = _embed(species, embed_atom)
    px, py, pz = pos[:, 0], pos[:, 1], pos[:, 2]

    eas, parts = [], []
    lo = 0
    for ec in CHUNKS:
        ic, jc = i[lo:lo + ec], j[lo:lo + ec]
        gi, gj, vx, vy, vz = _sc_gather(f, px, py, pz, ic, jc)
        vec = jnp.stack([vx, vy, vz, jnp.zeros((ec,), jnp.float32)], axis=-1)
        ea, msgs, mvx, mvy, mvz = _edges(
            gi, gj, vec, Wb1, bb1, Wb2, bb2,
            Ws1, bs1, Ws2, bs2, Wv1, bv1, Wv2, bv2)

        nr = ec // 128
        pad = (-nr) % 8
        j3 = jc.reshape(nr, 128)
        if pad:
            j3 = jnp.concatenate([j3, jnp.zeros((pad, 128), j.dtype)], axis=0)
        aggs, v0x = _sc_scatter2(j3, msgs, mvx)
        v0y, v0z = _sc_scatter2(j3, mvy, mvz)
        eas.append(ea)
        parts.append((aggs, v0x, v0y, v0z))
        lo += ec

    h0 = _nodes(f, [p[0][:N] for p in parts], Wh1, bh1, Wh2, bh2)
    v0 = jnp.stack(
        [sum(p[k] for p in parts)[:N] for k in (1, 2, 3)], axis=-1)
    ea = jnp.concatenate(eas, axis=0)
    return (h0, v0, ea)


# speedup vs baseline: 1.0333x; 1.0333x over previous
import functools

import jax
import jax.numpy as jnp
from jax import lax
from jax.experimental import pallas as pl
from jax.experimental.pallas import tpu as pltpu
from jax.experimental.pallas import tpu_sc as plsc

N = 10000
E = 320000
D = 128
ED = 16
S = 100

BN = 400
BE = 512

NC = 2
NS = 16
NW = NC * NS
G = 80

CHUNKS = (64000, 256000)


def _silu(x):
    return x * jax.nn.sigmoid(x)


def _embed_body(sp_ref, emb_ref, out_ref):
    sp = sp_ref[...]
    iota = lax.broadcasted_iota(jnp.int32, (BN, S), 1)
    onehot = (sp == iota).astype(jnp.float32)
    out_ref[...] = jnp.dot(onehot, emb_ref[...],
                           preferred_element_type=jnp.float32)


def _embed(species, embed_atom):
    return pl.pallas_call(
        _embed_body,
        grid=(N // BN,),
        in_specs=[
            pl.BlockSpec((BN, 1), lambda n: (n, 0)),
            pl.BlockSpec((S, D), lambda n: (0, 0)),
        ],
        out_specs=pl.BlockSpec((BN, D), lambda n: (n, 0)),
        out_shape=jax.ShapeDtypeStruct((N, D), jnp.float32),
    )(species.reshape(N, 1), embed_atom)


def _sc_gather_body(EPW, NIT,
                    f_hbm, px_hbm, py_hbm, pz_hbm, i_hbm, j_hbm,
                    gi_hbm, gj_hbm, vx_hbm, vy_hbm, vz_hbm,
                    idx_i, idx_j,
                    rows_i0, rows_j0, pbi0, pbj0, vb0,
                    rows_i1, rows_j1, pbi1, pbj1, vb1,
                    gsem0, gsem1, wsem0, wsem1):
    wid = lax.axis_index("s") * NC + lax.axis_index("c")
    base = wid * EPW
    pltpu.sync_copy(i_hbm.at[pl.ds(base, EPW)], idx_i)
    pltpu.sync_copy(j_hbm.at[pl.ds(base, EPW)], idx_j)

    sets = [(rows_i0, rows_j0, pbi0, pbj0, vb0, gsem0, wsem0),
            (rows_i1, rows_j1, pbi1, pbj1, vb1, gsem1, wsem1)]

    def fire_gathers(it, s):
        rows_i, rows_j, pbi, pbj, vb, gsem, wsem = s
        off = it * G
        ii = idx_i.at[pl.ds(off, G)]
        jj = idx_j.at[pl.ds(off, G)]
        pltpu.async_copy(f_hbm.at[ii], rows_i, gsem)
        pltpu.async_copy(f_hbm.at[jj], rows_j, gsem)
        pltpu.async_copy(px_hbm.at[ii], pbi.at[0], gsem)
        pltpu.async_copy(py_hbm.at[ii], pbi.at[1], gsem)
        pltpu.async_copy(pz_hbm.at[ii], pbi.at[2], gsem)
        pltpu.async_copy(px_hbm.at[jj], pbj.at[0], gsem)
        pltpu.async_copy(py_hbm.at[jj], pbj.at[1], gsem)
        pltpu.async_copy(pz_hbm.at[jj], pbj.at[2], gsem)

    def drain_gathers(s):
        rows_i, rows_j, pbi, pbj, vb, gsem, wsem = s
        dsrc = f_hbm.at[pl.ds(0, G)]
        pltpu.make_async_copy(dsrc, rows_i, gsem).wait()
        pltpu.make_async_copy(dsrc, rows_j, gsem).wait()
        dp = px_hbm.at[pl.ds(0, G)]
        for b in (pbi, pbj):
            for q in range(3):
                pltpu.make_async_copy(dp, b.at[q], gsem).wait()

    def compute_and_write(it, s):
        rows_i, rows_j, pbi, pbj, vb, gsem, wsem = s
        off = it * G
        for q in range(3):
            for r in range(G // 16):
                sl = pl.ds(r * 16, 16)
                vb[q, sl] = pbj[q, sl] - pbi[q, sl]
        pltpu.async_copy(rows_i, gi_hbm.at[pl.ds(base + off, G), :], wsem)
        pltpu.async_copy(rows_j, gj_hbm.at[pl.ds(base + off, G), :], wsem)
        pltpu.async_copy(vb.at[0], vx_hbm.at[pl.ds(base + off, G)], wsem)
        pltpu.async_copy(vb.at[1], vy_hbm.at[pl.ds(base + off, G)], wsem)
        pltpu.async_copy(vb.at[2], vz_hbm.at[pl.ds(base + off, G)], wsem)

    def drain_writes(s):
        rows_i, rows_j, pbi, pbj, vb, gsem, wsem = s
        dsrc = f_hbm.at[pl.ds(0, G)]
        pltpu.make_async_copy(dsrc, rows_i, wsem).wait()
        pltpu.make_async_copy(dsrc, rows_j, wsem).wait()
        dp = px_hbm.at[pl.ds(0, G)]
        pltpu.make_async_copy(dp, vb.at[0], wsem).wait()
        pltpu.make_async_copy(dp, vb.at[1], wsem).wait()
        pltpu.make_async_copy(dp, vb.at[2], wsem).wait()

    fire_gathers(0, sets[0])

    def body(h, carry):
        k = 2 * h
        drain_gathers(sets[0])

        @pl.when(h > 0)
        def _():
            drain_writes(sets[1])

        fire_gathers(k + 1, sets[1])
        compute_and_write(k, sets[0])
        drain_gathers(sets[1])
        drain_writes(sets[0])

        @pl.when(k + 2 < NIT)
        def _():
            fire_gathers(k + 2, sets[0])

        compute_and_write(k + 1, sets[1])
        return carry

    lax.fori_loop(0, NIT // 2, body, 0)

    if NIT % 2:
        drain_gathers(sets[0])
        drain_writes(sets[1])
        compute_and_write(NIT - 1, sets[0])
        drain_writes(sets[0])
    else:
        drain_writes(sets[1])


def _sc_gather(f, px, py, pz, i, j):
    ec = i.shape[0]
    EPW = ec // NW
    NIT = EPW // G
    ev = jax.ShapeDtypeStruct((ec,), jnp.float32)
    return pl.kernel(
        functools.partial(_sc_gather_body, EPW, NIT),
        out_type=[jax.ShapeDtypeStruct((ec, D), jnp.float32),
                  jax.ShapeDtypeStruct((ec, D), jnp.float32),
                  ev, ev, ev],
        mesh=plsc.VectorSubcoreMesh(core_axis_name="c", subcore_axis_name="s"),
        scratch_types=[
            pltpu.VMEM((EPW,), jnp.int32),
            pltpu.VMEM((EPW,), jnp.int32),
            pltpu.VMEM((G, D), jnp.float32),
            pltpu.VMEM((G, D), jnp.float32),
            pltpu.VMEM((3, G), jnp.float32),
            pltpu.VMEM((3, G), jnp.float32),
            pltpu.VMEM((3, G), jnp.float32),
            pltpu.VMEM((G, D), jnp.float32),
            pltpu.VMEM((G, D), jnp.float32),
            pltpu.VMEM((3, G), jnp.float32),
            pltpu.VMEM((3, G), jnp.float32),
            pltpu.VMEM((3, G), jnp.float32),
            pltpu.SemaphoreType.DMA,
            pltpu.SemaphoreType.DMA,
            pltpu.SemaphoreType.DMA,
            pltpu.SemaphoreType.DMA,
        ],
    )(f, px, py, pz, i, j)


NP = 10240
NPT = NP // NS


def _sc_scatter_body(NG, NTAIL, GPT,
                     ja_hbm, msga_hbm, msgb_hbm, outa_hbm, outb_hbm,
                     idx8, rows0, rows1, acc, sem0, sem1):
    c = lax.axis_index("c")
    t = lax.axis_index("s")

    def zrow(r, carry):
        for l in range(8):
            rows0[r, pl.ds(l * 16, 16)] = jnp.zeros((16,), jnp.float32)
        return carry

    lax.fori_loop(0, 128, zrow, 0)
    for k in range(NPT // 128):
        pltpu.sync_copy(rows0, acc.at[pl.ds(t * NPT + k * 128, 128), :])
    plsc.subcore_barrier()

    ngroups = (NG - 1 - t) // NS + 1
    nrows = ngroups * 8

    def process(msg_hbm):
        def grow(r):
            return ((r // 8) * NS + t) * 8 + (r % 8)

        def start(r, buf, sem):
            return pltpu.async_copy(
                msg_hbm.at[pl.ds(grow(r) * 128, 128), :], buf, sem)

        def drain(buf, sem):
            pltpu.make_async_copy(msg_hbm.at[pl.ds(0, 128), :], buf, sem).wait()

        def stage_idx(r):
            g = (r // 8) * NS + t
            pltpu.sync_copy(ja_hbm.at[pl.ds(g * 8, 8), :], idx8)

        def scat(r, buf):
            pltpu.sync_copy(buf, acc.at[idx8.at[lax.rem(r, 8)]], add=True)

        start(0, rows0, sem0)
        start(1, rows1, sem1)

        def body(h, carry):
            r = 2 * h

            @pl.when(lax.rem(r, 8) == 0)
            def _():
                stage_idx(r)

            drain(rows0, sem0)
            scat(r, rows0)

            @pl.when(r + 2 < nrows)
            def _():
                start(r + 2, rows0, sem0)

            drain(rows1, sem1)
            scat(r + 1, rows1)

            @pl.when(r + 3 < nrows)
            def _():
                start(r + 3, rows1, sem1)

            return carry

        lax.fori_loop(0, nrows // 2, body, 0)

        if NTAIL:
            @pl.when(t == 0)
            def _():
                pltpu.sync_copy(ja_hbm.at[pl.ds(NG * 8, 8), :], idx8)
                for k in range(NTAIL):
                    pltpu.sync_copy(
                        msg_hbm.at[pl.ds((NG * 8 + k) * 128, 128), :], rows0)
                    pltpu.sync_copy(rows0, acc.at[idx8.at[k]], add=True)

    @pl.when(c == 0)
    def _():
        process(msga_hbm)

    @pl.when(c == 1)
    def _():
        process(msgb_hbm)

    plsc.subcore_barrier()
    sl = pl.ds(t * NPT, NPT)

    @pl.when(c == 0)
    def _():
        pltpu.sync_copy(acc.at[sl, :], outa_hbm.at[sl, :])

    @pl.when(c == 1)
    def _():
        pltpu.sync_copy(acc.at[sl, :], outb_hbm.at[sl, :])


def _sc_scatter2(j3, msga, msgb):
    nr = msga.shape[0] // 128
    NG = nr // 8
    NTAIL = nr - NG * 8
    GPT = (NG + NS - 1) // NS
    on = jax.ShapeDtypeStruct((NP, D), jnp.float32)
    return pl.kernel(
        functools.partial(_sc_scatter_body, NG, NTAIL, GPT),
        out_type=[on, on],
        mesh=plsc.VectorSubcoreMesh(core_axis_name="c", subcore_axis_name="s"),
        scratch_types=[
            pltpu.VMEM((8, 128), jnp.int32),
            pltpu.VMEM((128, D), jnp.float32),
            pltpu.VMEM((128, D), jnp.float32),
            pltpu.VMEM_SHARED((NP, D), jnp.float32),
            pltpu.SemaphoreType.DMA,
            pltpu.SemaphoreType.DMA,
        ],
    )(j3, msga, msgb)


def _edge_body(gi_ref, gj_ref, vec_ref,
               Wb1, bb1, Wb2, bb2,
               Ws1a, Ws1b, Ws1c, bs1, Ws2, bs2,
               Wv1a, Wv1b, Wv1c, bv1, Wv2, bv2,
               ea_ref, msgs_ref, mvx_ref, mvy_ref, mvz_ref):
    fi = gi_ref[...]
    fj = gj_ref[...]
    vec = vec_ref[...][:, :3]
    length = jnp.sqrt(jnp.sum(vec * vec, axis=1, keepdims=True))

    ea = jnp.dot(_silu(length * Wb1[...] + bb1[...]), Wb2[...],
                 preferred_element_type=jnp.float32) + bb2[...]
    ea_ref[...] = ea

    pre_s = (jnp.dot(fi, Ws1a[...], preferred_element_type=jnp.float32)
             + jnp.dot(fj, Ws1b[...], preferred_element_type=jnp.float32)
             + jnp.dot(ea, Ws1c[...], preferred_element_type=jnp.float32)
             + bs1[...])
    msgs_ref[...] = (jnp.dot(_silu(pre_s), Ws2[...],
                             preferred_element_type=jnp.float32) + bs2[...]) * fi

    pre_v = (jnp.dot(fi, Wv1a[...], preferred_element_type=jnp.float32)
             + jnp.dot(fj, Wv1b[...], preferred_element_type=jnp.float32)
             + jnp.dot(ea, Wv1c[...], preferred_element_type=jnp.float32)
             + bv1[...])
    mv = jnp.dot(_silu(pre_v), Wv2[...],
                 preferred_element_type=jnp.float32) + bv2[...]
    mvx_ref[...] = mv * vec[:, 0:1]
    mvy_ref[...] = mv * vec[:, 1:2]
    mvz_ref[...] = mv * vec[:, 2:3]


def _edges(gi, gj, vec, Wb1, bb1, Wb2, bb2, Ws1, bs1, Ws2, bs2, Wv1, bv1, Wv2, bv2):
    ec = gi.shape[0]
    full = lambda shape: pl.BlockSpec(shape, lambda e: (0, 0))
    wspecs = [
        full((1, ED)), full((1, ED)), full((ED, ED)), full((1, ED)),
        full((D, D)), full((D, D)), full((ED, D)), full((1, D)),
        full((D, D)), full((1, D)),
        full((D, D)), full((D, D)), full((ED, D)), full((1, D)),
        full((D, D)), full((1, D)),
    ]
    out_shapes = [
        jax.ShapeDtypeStruct((ec, ED), jnp.float32),
        jax.ShapeDtypeStruct((ec, D), jnp.float32),
        jax.ShapeDtypeStruct((ec, D), jnp.float32),
        jax.ShapeDtypeStruct((ec, D), jnp.float32),
        jax.ShapeDtypeStruct((ec, D), jnp.float32),
    ]
    out_specs = [
        pl.BlockSpec((BE, ED), lambda e: (e, 0)),
        pl.BlockSpec((BE, D), lambda e: (e, 0)),
        pl.BlockSpec((BE, D), lambda e: (e, 0)),
        pl.BlockSpec((BE, D), lambda e: (e, 0)),
        pl.BlockSpec((BE, D), lambda e: (e, 0)),
    ]
    return pl.pallas_call(
        _edge_body,
        grid=(ec // BE,),
        in_specs=[pl.BlockSpec((BE, D), lambda e: (e, 0)),
                  pl.BlockSpec((BE, D), lambda e: (e, 0)),
                  pl.BlockSpec((BE, 4), lambda e: (e, 0))] + wspecs,
        out_specs=out_specs,
        out_shape=out_shapes,
    )(gi, gj, vec,
      Wb1, bb1.reshape(1, ED), Wb2, bb2.reshape(1, ED),
      Ws1[:D], Ws1[D:2 * D], Ws1[2 * D:], bs1.reshape(1, D), Ws2, bs2.reshape(1, D),
      Wv1[:D], Wv1[D:2 * D], Wv1[2 * D:], bv1.reshape(1, D), Wv2, bv2.reshape(1, D))


def _node_body(f_ref, *rest):
    naggs = len(rest) - 6
    aggs = rest[:naggs]
    Wh1a, Wh1b, bh1, Wh2, bh2, out_ref = rest[naggs:]
    agg = aggs[0][...]
    for a in aggs[1:]:
        agg = agg + a[...]
    pre = (jnp.dot(f_ref[...], Wh1a[...], preferred_element_type=jnp.float32)
           + jnp.dot(agg, Wh1b[...], preferred_element_type=jnp.float32)
           + bh1[...])
    out_ref[...] = jnp.dot(_silu(pre), Wh2[...],
                           preferred_element_type=jnp.float32) + bh2[...]


def _nodes(f, aggs, Wh1, bh1, Wh2, bh2):
    full = lambda shape: pl.BlockSpec(shape, lambda n: (0, 0))
    nspec = pl.BlockSpec((BN, D), lambda n: (n, 0))
    return pl.pallas_call(
        _node_body,
        grid=(N // BN,),
        in_specs=[nspec] + [nspec] * len(aggs) + [
            full((D, D)), full((D, D)), full((1, D)),
            full((D, D)), full((1, D))],
        out_specs=pl.BlockSpec((BN, D), lambda n: (n, 0)),
        out_shape=jax.ShapeDtypeStruct((N, D), jnp.float32),
    )(f, *aggs, Wh1[:D], Wh1[D:], bh1.reshape(1, D),
      Wh2, bh2.reshape(1, D))


def kernel(species, pos, edge_index, embed_atom,
           Wb1, bb1, Wb2, bb2,
           Ws1, bs1, Ws2, bs2,
           Wh1, bh1, Wh2, bh2,
           Wv1, bv1, Wv2, bv2):
    i = edge_index[0]
    j = edge_index[1]

    f = _embed(species, embed_atom)
    px, py, pz = pos[:, 0], pos[:, 1], pos[:, 2]

    eas, parts = [], []
    lo = 0
    for ec in CHUNKS:
        ic, jc = i[lo:lo + ec], j[lo:lo + ec]
        gi, gj, vx, vy, vz = _sc_gather(f, px, py, pz, ic, jc)
        vec = jnp.stack([vx, vy, vz, jnp.zeros((ec,), jnp.float32)], axis=-1)
        ea, msgs, mvx, mvy, mvz = _edges(
            gi, gj, vec, Wb1, bb1, Wb2, bb2,
            Ws1, bs1, Ws2, bs2, Wv1, bv1, Wv2, bv2)

        nr = ec // 128
        pad = (-nr) % 8
        j3 = jc.reshape(nr, 128)
        if pad:
            j3 = jnp.concatenate([j3, jnp.zeros((pad, 128), j.dtype)], axis=0)
        aggs, v0x = _sc_scatter2(j3, msgs, mvx)
        v0y, v0z = _sc_scatter2(j3, mvy, mvz)
        eas.append(ea)
        parts.append((aggs, v0x, v0y, v0z))
        lo += ec

    h0 = _nodes(f, [p[0][:N] for p in parts], Wh1, bh1, Wh2, bh2)
    v0 = jnp.stack(
        [sum(p[k] for p in parts)[:N] for k in (1, 2, 3)], axis=-1)
    ea = jnp.concatenate(eas, axis=0)
    return (h0, v0, ea)


# speedup vs baseline: 1.0876x; 1.0525x over previous
import functools

import jax
import jax.numpy as jnp
from jax import lax
from jax.experimental import pallas as pl
from jax.experimental.pallas import tpu as pltpu
from jax.experimental.pallas import tpu_sc as plsc

N = 10000
E = 320000
D = 128
ED = 16
S = 100

BN = 400
BE = 640

NC = 2
NS = 16
NW = NC * NS
G = 80

CHUNKS = (64000, 128000, 128000)


def _silu(x):
    return x * jax.nn.sigmoid(x)


def _embed_body(sp_ref, emb_ref, out_ref):
    sp = sp_ref[...]
    iota = lax.broadcasted_iota(jnp.int32, (BN, S), 1)
    onehot = (sp == iota).astype(jnp.float32)
    out_ref[...] = jnp.dot(onehot, emb_ref[...],
                           preferred_element_type=jnp.float32)


def _embed(species, embed_atom):
    return pl.pallas_call(
        _embed_body,
        grid=(N // BN,),
        in_specs=[
            pl.BlockSpec((BN, 1), lambda n: (n, 0)),
            pl.BlockSpec((S, D), lambda n: (0, 0)),
        ],
        out_specs=pl.BlockSpec((BN, D), lambda n: (n, 0)),
        out_shape=jax.ShapeDtypeStruct((N, D), jnp.float32),
    )(species.reshape(N, 1), embed_atom)


def _sc_gather_body(EPW, NIT,
                    f_hbm, px_hbm, py_hbm, pz_hbm, i_hbm, j_hbm,
                    gi_hbm, gj_hbm, vx_hbm, vy_hbm, vz_hbm,
                    idx_i, idx_j,
                    rows_i0, rows_j0, pbi0, pbj0, vb0,
                    rows_i1, rows_j1, pbi1, pbj1, vb1,
                    gsem0, gsem1, wsem0, wsem1):
    wid = lax.axis_index("s") * NC + lax.axis_index("c")
    base = wid * EPW
    pltpu.sync_copy(i_hbm.at[pl.ds(base, EPW)], idx_i)
    pltpu.sync_copy(j_hbm.at[pl.ds(base, EPW)], idx_j)

    sets = [(rows_i0, rows_j0, pbi0, pbj0, vb0, gsem0, wsem0),
            (rows_i1, rows_j1, pbi1, pbj1, vb1, gsem1, wsem1)]

    def fire_gathers(it, s):
        rows_i, rows_j, pbi, pbj, vb, gsem, wsem = s
        off = it * G
        ii = idx_i.at[pl.ds(off, G)]
        jj = idx_j.at[pl.ds(off, G)]
        pltpu.async_copy(f_hbm.at[ii], rows_i, gsem)
        pltpu.async_copy(f_hbm.at[jj], rows_j, gsem)
        pltpu.async_copy(px_hbm.at[ii], pbi.at[0], gsem)
        pltpu.async_copy(py_hbm.at[ii], pbi.at[1], gsem)
        pltpu.async_copy(pz_hbm.at[ii], pbi.at[2], gsem)
        pltpu.async_copy(px_hbm.at[jj], pbj.at[0], gsem)
        pltpu.async_copy(py_hbm.at[jj], pbj.at[1], gsem)
        pltpu.async_copy(pz_hbm.at[jj], pbj.at[2], gsem)

    def drain_gathers(s):
        rows_i, rows_j, pbi, pbj, vb, gsem, wsem = s
        dsrc = f_hbm.at[pl.ds(0, G)]
        pltpu.make_async_copy(dsrc, rows_i, gsem).wait()
        pltpu.make_async_copy(dsrc, rows_j, gsem).wait()
        dp = px_hbm.at[pl.ds(0, G)]
        for b in (pbi, pbj):
            for q in range(3):
                pltpu.make_async_copy(dp, b.at[q], gsem).wait()

    def compute_and_write(it, s):
        rows_i, rows_j, pbi, pbj, vb, gsem, wsem = s
        off = it * G
        for q in range(3):
            for r in range(G // 16):
                sl = pl.ds(r * 16, 16)
                vb[q, sl] = pbj[q, sl] - pbi[q, sl]
        pltpu.async_copy(rows_i, gi_hbm.at[pl.ds(base + off, G), :], wsem)
        pltpu.async_copy(rows_j, gj_hbm.at[pl.ds(base + off, G), :], wsem)
        pltpu.async_copy(vb.at[0], vx_hbm.at[pl.ds(base + off, G)], wsem)
        pltpu.async_copy(vb.at[1], vy_hbm.at[pl.ds(base + off, G)], wsem)
        pltpu.async_copy(vb.at[2], vz_hbm.at[pl.ds(base + off, G)], wsem)

    def drain_writes(s):
        rows_i, rows_j, pbi, pbj, vb, gsem, wsem = s
        dsrc = f_hbm.at[pl.ds(0, G)]
        pltpu.make_async_copy(dsrc, rows_i, wsem).wait()
        pltpu.make_async_copy(dsrc, rows_j, wsem).wait()
        dp = px_hbm.at[pl.ds(0, G)]
        pltpu.make_async_copy(dp, vb.at[0], wsem).wait()
        pltpu.make_async_copy(dp, vb.at[1], wsem).wait()
        pltpu.make_async_copy(dp, vb.at[2], wsem).wait()

    fire_gathers(0, sets[0])

    def body(h, carry):
        k = 2 * h
        drain_gathers(sets[0])

        @pl.when(h > 0)
        def _():
            drain_writes(sets[1])

        fire_gathers(k + 1, sets[1])
        compute_and_write(k, sets[0])
        drain_gathers(sets[1])
        drain_writes(sets[0])

        @pl.when(k + 2 < NIT)
        def _():
            fire_gathers(k + 2, sets[0])

        compute_and_write(k + 1, sets[1])
        return carry

    lax.fori_loop(0, NIT // 2, body, 0)

    if NIT % 2:
        drain_gathers(sets[0])
        drain_writes(sets[1])
        compute_and_write(NIT - 1, sets[0])
        drain_writes(sets[0])
    else:
        drain_writes(sets[1])


def _sc_gather(f, px, py, pz, i, j):
    ec = i.shape[0]
    EPW = ec // NW
    NIT = EPW // G
    ev = jax.ShapeDtypeStruct((ec,), jnp.float32)
    return pl.kernel(
        functools.partial(_sc_gather_body, EPW, NIT),
        out_type=[jax.ShapeDtypeStruct((ec, D), jnp.float32),
                  jax.ShapeDtypeStruct((ec, D), jnp.float32),
                  ev, ev, ev],
        mesh=plsc.VectorSubcoreMesh(core_axis_name="c", subcore_axis_name="s"),
        scratch_types=[
            pltpu.VMEM((EPW,), jnp.int32),
            pltpu.VMEM((EPW,), jnp.int32),
            pltpu.VMEM((G, D), jnp.float32),
            pltpu.VMEM((G, D), jnp.float32),
            pltpu.VMEM((3, G), jnp.float32),
            pltpu.VMEM((3, G), jnp.float32),
            pltpu.VMEM((3, G), jnp.float32),
            pltpu.VMEM((G, D), jnp.float32),
            pltpu.VMEM((G, D), jnp.float32),
            pltpu.VMEM((3, G), jnp.float32),
            pltpu.VMEM((3, G), jnp.float32),
            pltpu.VMEM((3, G), jnp.float32),
            pltpu.SemaphoreType.DMA,
            pltpu.SemaphoreType.DMA,
            pltpu.SemaphoreType.DMA,
            pltpu.SemaphoreType.DMA,
        ],
    )(f, px, py, pz, i, j)


NP = 10240
NPT = NP // NS


def _sc_scatter_body(NG, NTAIL, GPT,
                     ja_hbm, msga_hbm, msgb_hbm, outa_hbm, outb_hbm,
                     idx8, rows0, rows1, acc, sem0, sem1):
    c = lax.axis_index("c")
    t = lax.axis_index("s")

    def zrow(r, carry):
        for l in range(8):
            rows0[r, pl.ds(l * 16, 16)] = jnp.zeros((16,), jnp.float32)
        return carry

    lax.fori_loop(0, 128, zrow, 0)
    for k in range(NPT // 128):
        pltpu.sync_copy(rows0, acc.at[pl.ds(t * NPT + k * 128, 128), :])
    plsc.subcore_barrier()

    ngroups = (NG - 1 - t) // NS + 1
    nrows = ngroups * 8

    def process(msg_hbm):
        def grow(r):
            return ((r // 8) * NS + t) * 8 + (r % 8)

        def start(r, buf, sem):
            return pltpu.async_copy(
                msg_hbm.at[pl.ds(grow(r) * 128, 128), :], buf, sem)

        def drain(buf, sem):
            pltpu.make_async_copy(msg_hbm.at[pl.ds(0, 128), :], buf, sem).wait()

        def stage_idx(r):
            g = (r // 8) * NS + t
            pltpu.sync_copy(ja_hbm.at[pl.ds(g * 8, 8), :], idx8)

        def scat(r, buf):
            pltpu.sync_copy(buf, acc.at[idx8.at[lax.rem(r, 8)]], add=True)

        start(0, rows0, sem0)
        start(1, rows1, sem1)

        def body(h, carry):
            r = 2 * h

            @pl.when(lax.rem(r, 8) == 0)
            def _():
                stage_idx(r)

            drain(rows0, sem0)
            scat(r, rows0)

            @pl.when(r + 2 < nrows)
            def _():
                start(r + 2, rows0, sem0)

            drain(rows1, sem1)
            scat(r + 1, rows1)

            @pl.when(r + 3 < nrows)
            def _():
                start(r + 3, rows1, sem1)

            return carry

        lax.fori_loop(0, nrows // 2, body, 0)

        if NTAIL:
            @pl.when(t == 0)
            def _():
                pltpu.sync_copy(ja_hbm.at[pl.ds(NG * 8, 8), :], idx8)
                for k in range(NTAIL):
                    pltpu.sync_copy(
                        msg_hbm.at[pl.ds((NG * 8 + k) * 128, 128), :], rows0)
                    pltpu.sync_copy(rows0, acc.at[idx8.at[k]], add=True)

    @pl.when(c == 0)
    def _():
        process(msga_hbm)

    @pl.when(c == 1)
    def _():
        process(msgb_hbm)

    plsc.subcore_barrier()
    sl = pl.ds(t * NPT, NPT)

    @pl.when(c == 0)
    def _():
        pltpu.sync_copy(acc.at[sl, :], outa_hbm.at[sl, :])

    @pl.when(c == 1)
    def _():
        pltpu.sync_copy(acc.at[sl, :], outb_hbm.at[sl, :])


def _sc_scatter2(j3, msga, msgb):
    nr = msga.shape[0] // 128
    NG = nr // 8
    NTAIL = nr - NG * 8
    GPT = (NG + NS - 1) // NS
    on = jax.ShapeDtypeStruct((NP, D), jnp.float32)
    return pl.kernel(
        functools.partial(_sc_scatter_body, NG, NTAIL, GPT),
        out_type=[on, on],
        mesh=plsc.VectorSubcoreMesh(core_axis_name="c", subcore_axis_name="s"),
        scratch_types=[
            pltpu.VMEM((8, 128), jnp.int32),
            pltpu.VMEM((128, D), jnp.float32),
            pltpu.VMEM((128, D), jnp.float32),
            pltpu.VMEM_SHARED((NP, D), jnp.float32),
            pltpu.SemaphoreType.DMA,
            pltpu.SemaphoreType.DMA,
        ],
    )(j3, msga, msgb)


def _edge_body(gi_ref, gj_ref, vec_ref,
               Wb1, bb1, Wb2, bb2,
               Ws1a, Ws1b, Ws1c, bs1, Ws2, bs2,
               Wv1a, Wv1b, Wv1c, bv1, Wv2, bv2,
               ea_ref, msgs_ref, mvx_ref, mvy_ref, mvz_ref):
    fi = gi_ref[...]
    fj = gj_ref[...]
    vec = vec_ref[...][:, :3]
    length = jnp.sqrt(jnp.sum(vec * vec, axis=1, keepdims=True))

    ea = jnp.dot(_silu(length * Wb1[...] + bb1[...]), Wb2[...],
                 preferred_element_type=jnp.float32) + bb2[...]
    ea_ref[...] = ea

    pre_s = (jnp.dot(fi, Ws1a[...], preferred_element_type=jnp.float32)
             + jnp.dot(fj, Ws1b[...], preferred_element_type=jnp.float32)
             + jnp.dot(ea, Ws1c[...], preferred_element_type=jnp.float32)
             + bs1[...])
    msgs_ref[...] = (jnp.dot(_silu(pre_s), Ws2[...],
                             preferred_element_type=jnp.float32) + bs2[...]) * fi

    pre_v = (jnp.dot(fi, Wv1a[...], preferred_element_type=jnp.float32)
             + jnp.dot(fj, Wv1b[...], preferred_element_type=jnp.float32)
             + jnp.dot(ea, Wv1c[...], preferred_element_type=jnp.float32)
             + bv1[...])
    mv = jnp.dot(_silu(pre_v), Wv2[...],
                 preferred_element_type=jnp.float32) + bv2[...]
    mvx_ref[...] = mv * vec[:, 0:1]
    mvy_ref[...] = mv * vec[:, 1:2]
    mvz_ref[...] = mv * vec[:, 2:3]


def _edges(gi, gj, vec, Wb1, bb1, Wb2, bb2, Ws1, bs1, Ws2, bs2, Wv1, bv1, Wv2, bv2):
    ec = gi.shape[0]
    full = lambda shape: pl.BlockSpec(shape, lambda e: (0, 0))
    wspecs = [
        full((1, ED)), full((1, ED)), full((ED, ED)), full((1, ED)),
        full((D, D)), full((D, D)), full((ED, D)), full((1, D)),
        full((D, D)), full((1, D)),
        full((D, D)), full((D, D)), full((ED, D)), full((1, D)),
        full((D, D)), full((1, D)),
    ]
    out_shapes = [
        jax.ShapeDtypeStruct((ec, ED), jnp.float32),
        jax.ShapeDtypeStruct((ec, D), jnp.float32),
        jax.ShapeDtypeStruct((ec, D), jnp.float32),
        jax.ShapeDtypeStruct((ec, D), jnp.float32),
        jax.ShapeDtypeStruct((ec, D), jnp.float32),
    ]
    out_specs = [
        pl.BlockSpec((BE, ED), lambda e: (e, 0)),
        pl.BlockSpec((BE, D), lambda e: (e, 0)),
        pl.BlockSpec((BE, D), lambda e: (e, 0)),
        pl.BlockSpec((BE, D), lambda e: (e, 0)),
        pl.BlockSpec((BE, D), lambda e: (e, 0)),
    ]
    return pl.pallas_call(
        _edge_body,
        grid=(ec // BE,),
        in_specs=[pl.BlockSpec((BE, D), lambda e: (e, 0)),
                  pl.BlockSpec((BE, D), lambda e: (e, 0)),
                  pl.BlockSpec((BE, 4), lambda e: (e, 0))] + wspecs,
        out_specs=out_specs,
        out_shape=out_shapes,
    )(gi, gj, vec,
      Wb1, bb1.reshape(1, ED), Wb2, bb2.reshape(1, ED),
      Ws1[:D], Ws1[D:2 * D], Ws1[2 * D:], bs1.reshape(1, D), Ws2, bs2.reshape(1, D),
      Wv1[:D], Wv1[D:2 * D], Wv1[2 * D:], bv1.reshape(1, D), Wv2, bv2.reshape(1, D))


def _node_body(f_ref, *rest):
    naggs = len(rest) - 6
    aggs = rest[:naggs]
    Wh1a, Wh1b, bh1, Wh2, bh2, out_ref = rest[naggs:]
    agg = aggs[0][...]
    for a in aggs[1:]:
        agg = agg + a[...]
    pre = (jnp.dot(f_ref[...], Wh1a[...], preferred_element_type=jnp.float32)
           + jnp.dot(agg, Wh1b[...], preferred_element_type=jnp.float32)
           + bh1[...])
    out_ref[...] = jnp.dot(_silu(pre), Wh2[...],
                           preferred_element_type=jnp.float32) + bh2[...]


def _nodes(f, aggs, Wh1, bh1, Wh2, bh2):
    full = lambda shape: pl.BlockSpec(shape, lambda n: (0, 0))
    nspec = pl.BlockSpec((BN, D), lambda n: (n, 0))
    return pl.pallas_call(
        _node_body,
        grid=(N // BN,),
        in_specs=[nspec] + [nspec] * len(aggs) + [
            full((D, D)), full((D, D)), full((1, D)),
            full((D, D)), full((1, D))],
        out_specs=pl.BlockSpec((BN, D), lambda n: (n, 0)),
        out_shape=jax.ShapeDtypeStruct((N, D), jnp.float32),
    )(f, *aggs, Wh1[:D], Wh1[D:], bh1.reshape(1, D),
      Wh2, bh2.reshape(1, D))


def kernel(species, pos, edge_index, embed_atom,
           Wb1, bb1, Wb2, bb2,
           Ws1, bs1, Ws2, bs2,
           Wh1, bh1, Wh2, bh2,
           Wv1, bv1, Wv2, bv2):
    i = edge_index[0]
    j = edge_index[1]

    f = _embed(species, embed_atom)
    px, py, pz = pos[:, 0], pos[:, 1], pos[:, 2]

    eas, parts = [], []
    lo = 0
    for ec in CHUNKS:
        ic, jc = i[lo:lo + ec], j[lo:lo + ec]
        gi, gj, vx, vy, vz = _sc_gather(f, px, py, pz, ic, jc)
        vec = jnp.stack([vx, vy, vz, jnp.zeros((ec,), jnp.float32)], axis=-1)
        ea, msgs, mvx, mvy, mvz = _edges(
            gi, gj, vec, Wb1, bb1, Wb2, bb2,
            Ws1, bs1, Ws2, bs2, Wv1, bv1, Wv2, bv2)

        nr = ec // 128
        pad = (-nr) % 8
        j3 = jc.reshape(nr, 128)
        if pad:
            j3 = jnp.concatenate([j3, jnp.zeros((pad, 128), j.dtype)], axis=0)
        aggs, v0x = _sc_scatter2(j3, msgs, mvx)
        v0y, v0z = _sc_scatter2(j3, mvy, mvz)
        eas.append(ea)
        parts.append((aggs, v0x, v0y, v0z))
        lo += ec

    h0 = _nodes(f, [p[0][:N] for p in parts], Wh1, bh1, Wh2, bh2)
    v0 = jnp.stack(
        [sum(p[k] for p in parts)[:N] for k in (1, 2, 3)], axis=-1)
    ea = jnp.concatenate(eas, axis=0)
    return (h0, v0, ea)
